# Initial kernel scaffold; baseline (speedup 1.0000x reference)
import jax
import jax.numpy as jnp
from jax.experimental import pallas as pl


def kernel(pos, batch, fc1_w1, fc1_w2, U1, V1, C1, fc2_w1, fc2_w2, U2, V2, C2, fc3_w1, fc3_w2, U3, V3, C3):
    raise NotImplementedError("write your pallas kernel here")


# speedup vs baseline: 13.3685x; 13.3685x over previous
import numpy as np
import jax
import jax.numpy as jnp
from jax.experimental import pallas as pl
from jax.experimental.pallas import tpu as pltpu

_NN = 3.8
_R2 = 6.2 ** 2
_N = 10000
_G = 100
_T = 512
_W = 3 * _T
_NPAD = 10240
_NT = _NPAD // _T
_NWIN = _NPAD + 2 * _T


def _sh16(ux, uy, uz):
    c1 = 3.0 ** 0.5
    c2 = 15.0 ** 0.5
    return [
        jnp.ones_like(ux),
        c1 * uy, c1 * uz, c1 * ux,
        c2 * ux * uy, c2 * uy * uz,
        (5.0 ** 0.5) / 2.0 * (3.0 * uz * uz - 1.0),
        c2 * ux * uz, c2 / 2.0 * (ux * ux - uy * uy),
        (35.0 / 8.0) ** 0.5 * uy * (3.0 * ux * ux - uy * uy),
        (105.0 ** 0.5) * ux * uy * uz,
        (21.0 / 8.0) ** 0.5 * uy * (5.0 * uz * uz - 1.0),
        (7.0 ** 0.5) / 2.0 * uz * (5.0 * uz * uz - 3.0),
        (21.0 / 8.0) ** 0.5 * ux * (5.0 * uz * uz - 1.0),
        (105.0 ** 0.5) / 2.0 * uz * (ux * ux - uy * uy),
        (35.0 / 8.0) ** 0.5 * ux * (ux * ux - 3.0 * uy * uy),
    ]


def _soft1h(d):
    out = []
    for v in (1.0, 1.5, 2.0):
        diff = (d - v) / 0.5
        m = jnp.abs(diff) < 1.0
        denom = jnp.where(m, 1.0 - diff * diff, 1.0)
        out.append(jnp.where(m, 1.14136 * float(np.e ** 2) * jnp.exp(-1.0 / denom), 0.0)
                   * (3.0 ** 0.5))
    return out


def _pair_geom(posr_ref, batchr_ref, dpos_ref, dbatch_ref, base, cb, r0, rows,
               width):
    sx = posr_ref[0:1, pl.ds(cb, width)]
    sy = posr_ref[1:2, pl.ds(cb, width)]
    sz = posr_ref[2:3, pl.ds(cb, width)]
    dx = dpos_ref[pl.ds(r0, rows), 0:1]
    dy = dpos_ref[pl.ds(r0, rows), 1:2]
    dz = dpos_ref[pl.ds(r0, rows), 2:3]
    n2s = sx * sx + sy * sy + sz * sz
    n2d = dx * dx + dy * dy + dz * dz
    dots = dx * sx + dy * sy + dz * sz
    d2 = (n2d + n2s) - 2.0 * dots
    bs = batchr_ref[0:1, pl.ds(cb, width)]
    bd = dbatch_ref[pl.ds(r0, rows), 0:1]
    ii = jax.lax.broadcasted_iota(jnp.int32, (rows, width), 1) + cb
    jj = jax.lax.broadcasted_iota(jnp.int32, (rows, width), 0) + (base + _T + r0)
    mask = (d2 < _R2) & (bs == bd) & (ii != jj)
    evx = sx - dx
    evy = sy - dy
    evz = sz - dz
    nrm = jnp.sqrt(evx * evx + evy * evy + evz * evz)
    inv = 1.0 / jnp.maximum(nrm, 1e-9)
    return mask, evx * inv, evy * inv, evz * inv, nrm


def _l0_kernel(posr_ref, batchr_ref, dpos_ref, dbatch_ref, out_ref):
    i = pl.program_id(0)
    base = i * _T
    R = 32

    def body(k, _):
        mask, ux, uy, uz, _ = _pair_geom(posr_ref, batchr_ref, dpos_ref,
                                         dbatch_ref, base, base, k * R, R, _W)
        sh = _sh16(ux, uy, uz)
        scale = 1.0 / (_NN ** 0.5)
        for c in range(16):
            col = jnp.sum(jnp.where(mask, sh[c], 0.0), axis=1, keepdims=True)
            out_ref[pl.ds(k * R, R), c:c + 1] = col * scale
        return 0

    jax.lax.fori_loop(0, _T // R, body, 0)


def _edge_weights(mask, ux, uy, uz, nrm, w1_ref, w2_ref, v_ref, R, width):
    sh = jnp.stack(_sh16(ux, uy, uz), axis=-1).reshape(R * width, 16)
    es = jnp.stack([jnp.where(mask, e, 0.0) for e in _soft1h(nrm)],
                   axis=-1).reshape(R * width, 3)
    w = jnp.zeros((R * width, 64), jnp.float32)
    for hc in range(2):
        h = jnp.maximum(jnp.dot(es, w1_ref[:, hc * 128:(hc + 1) * 128],
                                preferred_element_type=jnp.float32), 0.0)
        w = w + jnp.dot(h, w2_ref[hc * 128:(hc + 1) * 128, :],
                        preferred_element_type=jnp.float32)
    b = jnp.dot(sh, v_ref[...].T, preferred_element_type=jnp.float32)
    return w * b


def _src_feats(xl_ref, xc_ref, xr_ref, u_ref):
    a = [jnp.dot(x_ref[...], u_ref[...].T, preferred_element_type=jnp.float32)
         for x_ref in (xl_ref, xc_ref, xr_ref)]
    return jnp.concatenate(a, axis=0)


def _conv_kernel(gate, pre_dim, posr_ref, batchr_ref, dpos_ref, dbatch_ref,
                 xl_ref, xc_ref, xr_ref, w1_ref, w2_ref, u_ref, v_ref, c_ref,
                 out_ref, acc_ref):
    i = pl.program_id(0)
    base = i * _T
    a = _src_feats(xl_ref, xc_ref, xr_ref, u_ref)
    R = 8

    def body(k, _):
        red = jnp.zeros((R, 64), jnp.float32)
        for c in range(3):
            mask, ux, uy, uz, nrm = _pair_geom(
                posr_ref, batchr_ref, dpos_ref, dbatch_ref,
                base, base + c * _T, k * R, R, _T)
            t = _edge_weights(mask, ux, uy, uz, nrm, w1_ref, w2_ref, v_ref,
                              R, _T)
            ac = a[c * _T:(c + 1) * _T, :]
            red = red + jnp.sum(t.reshape(R, _T, 64) * ac[None, :, :], axis=1)
        ef = jnp.dot(red, c_ref[...], preferred_element_type=jnp.float32)
        acc_ref[pl.ds(k * R, R), :] = ef * (1.0 / (_NN ** 0.5))
        return 0

    jax.lax.fori_loop(0, _T // R, body, 0)
    gate(acc_ref, out_ref)


def _gate1(acc_ref, out_ref):
    x = acc_ref[...]
    s = x[:, 0:32]
    g = x[:, 32:64]
    v = x[:, 64:160]
    i32 = jax.lax.broadcasted_iota(jnp.int32, (_T, 32), 1)
    act_s = jnp.where(i32 < 16, jnp.maximum(s, 0.0), jnp.abs(s))
    act_g = jnp.where((i32 // 8) % 2 == 0, jnp.maximum(g, 0.0), jnp.tanh(g))
    rk = jax.lax.broadcasted_iota(jnp.int32, (32, 96), 0)
    rc = jax.lax.broadcasted_iota(jnp.int32, (32, 96), 1)
    rep = (rc // 3 == rk).astype(jnp.float32)
    out_ref[:, 0:32] = act_s
    out_ref[:, 32:128] = v * jnp.dot(act_g, rep,
                                     preferred_element_type=jnp.float32)


def _gate2(acc_ref, out_ref):
    x = acc_ref[...]
    s = x[:, 0:32]
    g = x[:, 32:56]
    v = x[:, 56:128]
    i32 = jax.lax.broadcasted_iota(jnp.int32, (_T, 32), 1)
    act_s = jnp.where(i32 < 16, jnp.maximum(s, 0.0), jnp.abs(s))
    i24 = jax.lax.broadcasted_iota(jnp.int32, (_T, 24), 1)
    act_g = jnp.where((i24 // 6) % 2 == 0, jnp.maximum(g, 0.0), jnp.tanh(g))
    rk = jax.lax.broadcasted_iota(jnp.int32, (24, 72), 0)
    rc = jax.lax.broadcasted_iota(jnp.int32, (24, 72), 1)
    rep = (rc // 3 == rk).astype(jnp.float32)
    out_ref[:, 0:32] = act_s
    out_ref[:, 32:104] = v * jnp.dot(act_g, rep,
                                     preferred_element_type=jnp.float32)


def _conv3_kernel(posr_ref, batchr_ref, dpos_ref, dbatch_ref,
                  xl_ref, xc_ref, xr_ref, w1_ref, w2_ref, u_ref, v_ref, c_ref,
                  out_ref):
    i = pl.program_id(0)
    base = i * _T

    @pl.when(i == 0)
    def _init():
        out_ref[...] = jnp.zeros_like(out_ref)

    a = _src_feats(xl_ref, xc_ref, xr_ref, u_ref)
    R = 8
    gi = jax.lax.broadcasted_iota(jnp.int32, (R, 128), 1)

    def body(k, part):
        red = jnp.zeros((R, 64), jnp.float32)
        for c in range(3):
            mask, ux, uy, uz, nrm = _pair_geom(
                posr_ref, batchr_ref, dpos_ref, dbatch_ref,
                base, base + c * _T, k * R, R, _T)
            t = _edge_weights(mask, ux, uy, uz, nrm, w1_ref, w2_ref, v_ref,
                              R, _T)
            ac = a[c * _T:(c + 1) * _T, :]
            red = red + jnp.sum(t.reshape(R, _T, 64) * ac[None, :, :], axis=1)
        x3 = jnp.dot(red, c_ref[...], preferred_element_type=jnp.float32)
        x3 = x3 * (1.0 / (_NN ** 0.5))
        bd = dbatch_ref[pl.ds(k * R, R), 0:1]
        onz = jnp.where(gi == bd, x3, 0.0)
        return part + jnp.sum(onz, axis=0, keepdims=True)

    part = jax.lax.fori_loop(0, _T // R, body,
                             jnp.zeros((1, 128), jnp.float32))
    out_ref[...] = out_ref[...] + part

    @pl.when(i == _NT - 1)
    def _fin():
        out_ref[...] = jax.nn.sigmoid(out_ref[...] * 0.5)


def _full_spec(shape):
    return pl.BlockSpec(shape, lambda i: tuple(0 for _ in shape))


def kernel(pos, batch, fc1_w1, fc1_w2, U1, V1, C1, fc2_w1, fc2_w2, U2, V2, C2,
           fc3_w1, fc3_w2, U3, V3, C3):
    f32 = jnp.float32
    posp = jnp.zeros((_NPAD, 3), f32).at[: _N, :].set(pos.astype(f32))
    batp = jnp.full((_NPAD, 1), -1, jnp.int32).at[: _N, 0].set(batch)
    poswr = jnp.zeros((3, _NWIN), f32).at[:, _T:_T + _NPAD].set(posp.T)
    batwr = jnp.full((1, _NWIN), -1, jnp.int32).at[:, _T:_T + _NPAD].set(batp.T)

    cp = pltpu.CompilerParams(dimension_semantics=("arbitrary",))
    base_specs = [_full_spec((3, _NWIN)), _full_spec((1, _NWIN)),
                  pl.BlockSpec((_T, 3), lambda i: (i, 0)),
                  pl.BlockSpec((_T, 1), lambda i: (i, 0))]

    x0 = pl.pallas_call(
        _l0_kernel,
        grid=(_NT,),
        in_specs=list(base_specs),
        out_specs=pl.BlockSpec((_T, 16), lambda i: (i, 0)),
        out_shape=jax.ShapeDtypeStruct((_NPAD, 16), f32),
        compiler_params=cp,
    )(poswr, batwr, posp, batp)

    def xw_specs(fdim):
        return [pl.BlockSpec((_T, fdim), lambda i: (i, 0)),
                pl.BlockSpec((_T, fdim), lambda i: (i + 1, 0)),
                pl.BlockSpec((_T, fdim), lambda i: (i + 2, 0))]

    def conv(x, w1, w2, U, V, C, gate, pre_dim, out_dim):
        fdim = x.shape[1]
        xw = jnp.zeros((_NWIN, fdim), f32).at[_T:_T + _NPAD, :].set(x)
        fn = lambda *refs: _conv_kernel(gate, pre_dim, *refs)
        return pl.pallas_call(
            fn,
            grid=(_NT,),
            in_specs=base_specs + xw_specs(fdim) + [
                _full_spec(w1.shape), _full_spec(w2.shape),
                _full_spec(U.shape), _full_spec(V.shape),
                _full_spec(C.shape)],
            out_specs=pl.BlockSpec((_T, out_dim), lambda i: (i, 0)),
            out_shape=jax.ShapeDtypeStruct((_NPAD, out_dim), f32),
            scratch_shapes=[pltpu.VMEM((_T, pre_dim), f32)],
            compiler_params=cp,
        )(poswr, batwr, posp, batp, xw, xw, xw, w1, w2, U, V, C)

    s3 = 1.0 / np.sqrt(3.0)
    s256 = 1.0 / np.sqrt(256.0)
    x1 = conv(x0, fc1_w1 * s3, fc1_w2 * s256, U1, V1, C1, _gate1, 160, 128)
    x2 = conv(x1, fc2_w1 * s3, fc2_w2 * s256, U2, V2, C2, _gate2, 128, 104)

    xw2 = jnp.zeros((_NWIN, 104), f32).at[_T:_T + _NPAD, :].set(x2)
    out = pl.pallas_call(
        _conv3_kernel,
        grid=(_NT,),
        in_specs=base_specs + xw_specs(104) + [
            _full_spec(fc3_w1.shape), _full_spec(fc3_w2.shape),
            _full_spec(U3.shape), _full_spec(V3.shape),
            _full_spec(C3.shape)],
        out_specs=pl.BlockSpec((1, 128), lambda i: (0, 0)),
        out_shape=jax.ShapeDtypeStruct((1, 128), f32),
        compiler_params=cp,
    )(poswr, batwr, posp, batp, xw2, xw2, xw2,
      fc3_w1 * s3, fc3_w2 * s256, U3, V3, C3)

    return out[0, :_G].reshape(_G, 1)
